# baseline (speedup 1.0000x reference)
import functools

import jax
import jax.numpy as jnp
from jax import lax
from jax.experimental import pallas as pl
from jax.experimental.pallas import tpu as pltpu
from jax.experimental.pallas import tpu_sc as plsc

ROWS, COLS = 16384, 1024
NC, NS = 2, 16
NW = NC * NS
RPW = ROWS // NW
CHUNK = 32
NCHUNK = RPW // CHUNK
LANES = 16
VPR = COLS // LANES


def _sc_body(data_hbm, fac_hbm, out_hbm, buf, fac_v):
    wid = lax.axis_index("s") * NC + lax.axis_index("c")
    base = wid * RPW
    pltpu.sync_copy(fac_hbm.at[pl.ds(base, RPW)], fac_v)

    def chunk_body(c, _):
        rbase = base + c * CHUNK
        pltpu.sync_copy(data_hbm.at[pl.ds(rbase, CHUNK), :], buf)
        for g in range(CHUNK // LANES):
            fvec = 1.0 + fac_v[pl.ds(c * CHUNK + g * LANES, LANES)]
            for l in range(LANES):
                row = g * LANES + l
                f = fvec[l]

                def vec_body(j, _):
                    v = buf[row, pl.ds(j * LANES, LANES)]
                    buf[row, pl.ds(j * LANES, LANES)] = v * f
                    return 0

                lax.fori_loop(0, VPR, vec_body, 0, unroll=4)
        pltpu.sync_copy(buf, out_hbm.at[pl.ds(rbase, CHUNK), :])
        return 0

    lax.fori_loop(0, NCHUNK, chunk_body, 0)


_sc_call = functools.partial(
    pl.kernel,
    out_type=jax.ShapeDtypeStruct((ROWS, COLS), jnp.float32),
    mesh=plsc.VectorSubcoreMesh(core_axis_name="c", subcore_axis_name="s"),
    scratch_types=[
        pltpu.VMEM((CHUNK, COLS), jnp.float32),
        pltpu.VMEM((RPW,), jnp.float32),
    ],
)(_sc_body)


def kernel(data, applied_mask):
    fac = applied_mask.astype(jnp.float32)
    return _sc_call(data, fac)
